# baseline (speedup 1.0000x reference)
import jax
import jax.numpy as jnp
from jax.experimental import pallas as pl

N = 10000


def _bias_add_kernel(x_ref, b_ref, o_ref):
    o_ref[...] = x_ref[...] + b_ref[...]


def _sage_pool(x, src, dst, Wp, bp, Ws, Wn, bn):
    hp = jax.nn.relu(x @ Wp + bp)
    msg = jnp.take(hp, src, axis=0)
    agg = jax.ops.segment_max(msg, dst, num_segments=N)
    agg = jnp.where(jnp.isfinite(agg), agg, 0.0)
    return x @ Ws + agg @ Wn + bn


def _hetero_layer(x, e0, e1, Wp, bp, Ws, Wn, bn):
    o0 = _sage_pool(x, e0[0], e0[1], Wp[0], bp[0], Ws[0], Wn[0], bn[0])
    o1 = _sage_pool(x, e1[0], e1[1], Wp[1], bp[1], Ws[1], Wn[1], bn[1])
    return (o0 + o1) * 0.5


def kernel(x, edge_index_rel0, edge_index_rel1, dec_edge_index,
           Wp1, bp1, Ws1, Wn1, bn1,
           Wp2, bp2, Ws2, Wn2, bn2,
           Wp3, bp3, Ws3, Wn3, bn3,
           Wpred, bpred):
    h = jax.nn.relu(_hetero_layer(x, edge_index_rel0, edge_index_rel1, Wp1, bp1, Ws1, Wn1, bn1))
    h = jax.nn.relu(_hetero_layer(h, edge_index_rel0, edge_index_rel1, Wp2, bp2, Ws2, Wn2, bn2))
    h = _hetero_layer(h, edge_index_rel0, edge_index_rel1, Wp3, bp3, Ws3, Wn3, bn3)
    src = dec_edge_index[0]
    dst = dec_edge_index[1]
    feat = jnp.concatenate([jnp.take(h, src, axis=0), jnp.take(h, dst, axis=0)], axis=1)
    out = feat @ Wpred
    bias = jnp.broadcast_to(bpred[None, :], out.shape)
    blk = 8000
    return pl.pallas_call(
        _bias_add_kernel,
        grid=(out.shape[0] // blk,),
        in_specs=[
            pl.BlockSpec((blk, out.shape[1]), lambda i: (i, 0)),
            pl.BlockSpec((blk, out.shape[1]), lambda i: (i, 0)),
        ],
        out_specs=pl.BlockSpec((blk, out.shape[1]), lambda i: (i, 0)),
        out_shape=jax.ShapeDtypeStruct(out.shape, out.dtype),
    )(out, bias)


# speedup vs baseline: 1.5620x; 1.5620x over previous
import functools

import jax
import jax.numpy as jnp
from jax.experimental import pallas as pl
from jax.experimental.pallas import tpu as pltpu

N = 10000
E = 320000
D = 128

ROW_BLK = 2000
CHUNK = 512
KACC = 8
NEG = float("-inf")


def _hp_body(x_ref, w_ref, b_ref, o_ref):
    o_ref[...] = jax.nn.relu(
        jnp.dot(x_ref[...], w_ref[...], preferred_element_type=jnp.float32)
        + b_ref[...])


def _hp(x, w, b):
    return pl.pallas_call(
        _hp_body,
        grid=(N // ROW_BLK,),
        in_specs=[
            pl.BlockSpec((ROW_BLK, D), lambda i: (i, 0)),
            pl.BlockSpec((D, D), lambda i: (0, 0)),
            pl.BlockSpec((1, D), lambda i: (0, 0)),
        ],
        out_specs=pl.BlockSpec((ROW_BLK, D), lambda i: (i, 0)),
        out_shape=jax.ShapeDtypeStruct((N, D), jnp.float32),
    )(x, w, b)


def _combine_body(act, x_ref, a0_ref, a1_ref, wss_ref, wn0_ref, wn1_ref,
                  bnn_ref, o_ref):
    o = (jnp.dot(x_ref[...], wss_ref[...], preferred_element_type=jnp.float32)
         + jnp.dot(a0_ref[...], wn0_ref[...], preferred_element_type=jnp.float32)
         + jnp.dot(a1_ref[...], wn1_ref[...], preferred_element_type=jnp.float32)
         + bnn_ref[...]) * 0.5
    o_ref[...] = jax.nn.relu(o) if act else o


def _combine(x, a0, a1, wss, wn0, wn1, bnn, act):
    return pl.pallas_call(
        functools.partial(_combine_body, act),
        grid=(N // ROW_BLK,),
        in_specs=[
            pl.BlockSpec((ROW_BLK, D), lambda i: (i, 0)),
            pl.BlockSpec((ROW_BLK, D), lambda i: (i, 0)),
            pl.BlockSpec((ROW_BLK, D), lambda i: (i, 0)),
            pl.BlockSpec((D, D), lambda i: (0, 0)),
            pl.BlockSpec((D, D), lambda i: (0, 0)),
            pl.BlockSpec((D, D), lambda i: (0, 0)),
            pl.BlockSpec((1, D), lambda i: (0, 0)),
        ],
        out_specs=pl.BlockSpec((ROW_BLK, D), lambda i: (i, 0)),
        out_shape=jax.ShapeDtypeStruct((N, D), jnp.float32),
    )(x, a0, a1, wss, wn0, wn1, bnn)


def _segmax_body(src_ref, dst_ref, hp_ref, o_ref, *accs):
    step = pl.program_id(0)

    @pl.when(step == 0)
    def _init():
        for a in accs:
            a[...] = jnp.full((N, D), NEG, jnp.float32)

    def body(i, carry):
        for k in range(KACC):
            e = i * KACC + k
            s = src_ref[e]
            d = dst_ref[e]
            row = hp_ref[pl.ds(s, 1), :]
            a = accs[k]
            a[pl.ds(d, 1), :] = jnp.maximum(a[pl.ds(d, 1), :], row)
        return carry

    jax.lax.fori_loop(0, CHUNK // KACC, body, jnp.int32(0))

    @pl.when(step == pl.num_programs(0) - 1)
    def _fin():
        m = accs[0][...]
        for a in accs[1:]:
            m = jnp.maximum(m, a[...])
        o_ref[...] = jnp.where(jnp.isfinite(m), m, 0.0)


def _segmax(src, dst, hp):
    return pl.pallas_call(
        _segmax_body,
        grid=(E // CHUNK,),
        in_specs=[
            pl.BlockSpec((CHUNK,), lambda j: (j,), memory_space=pltpu.SMEM),
            pl.BlockSpec((CHUNK,), lambda j: (j,), memory_space=pltpu.SMEM),
            pl.BlockSpec((N, D), lambda j: (0, 0)),
        ],
        out_specs=pl.BlockSpec((N, D), lambda j: (0, 0)),
        out_shape=jax.ShapeDtypeStruct((N, D), jnp.float32),
        scratch_shapes=[pltpu.VMEM((N, D), jnp.float32) for _ in range(KACC)],
    )(src, dst, hp)


def _pred_mm_body(h_ref, wa_ref, wb_ref, a_ref, b_ref):
    a_ref[...] = jnp.dot(h_ref[...], wa_ref[...],
                         preferred_element_type=jnp.float32)
    b_ref[...] = jnp.dot(h_ref[...], wb_ref[...],
                         preferred_element_type=jnp.float32)


def _pred_mm(h, wa, wb):
    return pl.pallas_call(
        _pred_mm_body,
        grid=(N // ROW_BLK,),
        in_specs=[
            pl.BlockSpec((ROW_BLK, D), lambda i: (i, 0)),
            pl.BlockSpec((D, 8), lambda i: (0, 0)),
            pl.BlockSpec((D, 8), lambda i: (0, 0)),
        ],
        out_specs=(pl.BlockSpec((ROW_BLK, 8), lambda i: (i, 0)),
                   pl.BlockSpec((ROW_BLK, 8), lambda i: (i, 0))),
        out_shape=(jax.ShapeDtypeStruct((N, 8), jnp.float32),
                   jax.ShapeDtypeStruct((N, 8), jnp.float32)),
    )(h, wa, wb)


def _pred_gather_body(src_ref, dst_ref, a_ref, b_ref, bp_ref, o_ref):
    bp = bp_ref[...]

    def body(i, carry):
        s = src_ref[i]
        d = dst_ref[i]
        o_ref[pl.ds(i, 1), :] = a_ref[pl.ds(s, 1), :] + b_ref[pl.ds(d, 1), :] + bp
        return carry

    jax.lax.fori_loop(0, CHUNK, body, jnp.int32(0))


def _pred_gather(src, dst, a, b, bp):
    return pl.pallas_call(
        _pred_gather_body,
        grid=(E // CHUNK,),
        in_specs=[
            pl.BlockSpec((CHUNK,), lambda j: (j,), memory_space=pltpu.SMEM),
            pl.BlockSpec((CHUNK,), lambda j: (j,), memory_space=pltpu.SMEM),
            pl.BlockSpec((N, 8), lambda j: (0, 0)),
            pl.BlockSpec((N, 8), lambda j: (0, 0)),
            pl.BlockSpec((1, 8), lambda j: (0, 0)),
        ],
        out_specs=pl.BlockSpec((CHUNK, 8), lambda j: (j, 0)),
        out_shape=jax.ShapeDtypeStruct((E, 8), jnp.float32),
    )(src, dst, a, b, bp)


def _layer(x, e0, e1, Wp, bp, Ws, Wn, bn, act):
    aggs = []
    for r, e in ((0, e0), (1, e1)):
        hp = _hp(x, Wp[r], bp[r].reshape(1, D))
        aggs.append(_segmax(e[0], e[1], hp))
    wss = Ws[0] + Ws[1]
    bnn = (bn[0] + bn[1]).reshape(1, D)
    return _combine(x, aggs[0], aggs[1], wss, Wn[0], Wn[1], bnn, act)


def kernel(x, edge_index_rel0, edge_index_rel1, dec_edge_index,
           Wp1, bp1, Ws1, Wn1, bn1,
           Wp2, bp2, Ws2, Wn2, bn2,
           Wp3, bp3, Ws3, Wn3, bn3,
           Wpred, bpred):
    h = _layer(x, edge_index_rel0, edge_index_rel1, Wp1, bp1, Ws1, Wn1, bn1, True)
    h = _layer(h, edge_index_rel0, edge_index_rel1, Wp2, bp2, Ws2, Wn2, bn2, True)
    h = _layer(h, edge_index_rel0, edge_index_rel1, Wp3, bp3, Ws3, Wn3, bn3, False)

    wa = jnp.zeros((D, 8), jnp.float32).at[:, :2].set(Wpred[:D])
    wb = jnp.zeros((D, 8), jnp.float32).at[:, :2].set(Wpred[D:])
    bp8 = jnp.zeros((1, 8), jnp.float32).at[0, :2].set(bpred)
    a, b = _pred_mm(h, wa, wb)
    out8 = _pred_gather(dec_edge_index[0], dec_edge_index[1], a, b, bp8)
    return out8[:, :2]
